# Initial kernel scaffold; baseline (speedup 1.0000x reference)
import jax
import jax.numpy as jnp
from jax.experimental import pallas as pl


def kernel(input_features, incidence_matrix, W_enc, b_enc, W_attr, b_attr, W_conv, att, b_conv, W_out, b_out):
    raise NotImplementedError("write your pallas kernel here")


# speedup vs baseline: 5453.6386x; 5453.6386x over previous
import jax
import jax.numpy as jnp
from jax.experimental import pallas as pl

_DNT = (((1,), (1,)), ((), ()))


def _mushin_body(inp_ref, incT_ref, wenc_ref, benc_ref, wattr_ref, battr_ref,
                 wc0, wc1, wc2, ai0, ai1, ai2, aj0, aj1, aj2,
                 bc0, bc1, bc2, wo0, wo1, wo2, bout_ref, out_ref):
    f32 = jnp.float32

    incT = incT_ref[...]
    maskT = incT > 0.0

    x = jnp.dot(inp_ref[...], wenc_ref[...], preferred_element_type=f32)
    x = jnp.maximum(x + benc_ref[...], 0.0)

    he = jnp.dot(incT, wattr_ref[...], preferred_element_type=f32)
    he = he + battr_ref[...]

    deg_n = jnp.sum(incT, axis=0, keepdims=True)
    inv_dn = jnp.where(deg_n > 0.0, 1.0 / deg_n, 0.0)
    deg_e = jnp.sum(incT, axis=1, keepdims=True)
    inv_de = jnp.where(deg_e > 0.0, 1.0 / deg_e, 0.0)

    res = None
    for wc, ai, aj, bc, wo in ((wc0, ai0, aj0, bc0, wo0),
                               (wc1, ai1, aj1, bc1, wo1),
                               (wc2, ai2, aj2, bc2, wo2)):
        xl = jnp.dot(x, wc[...], preferred_element_type=f32)
        ea = jnp.dot(he, wc[...], preferred_element_type=f32)
        a_i = jax.lax.dot_general(ai[...], xl, _DNT,
                                  preferred_element_type=f32)
        a_e = jax.lax.dot_general(ea, aj[...], _DNT,
                                  preferred_element_type=f32)
        logit = a_i + a_e
        logit = jnp.where(logit >= 0.0, logit, 0.2 * logit)
        lmask = jnp.where(maskT, logit, -1e30)
        amax = jnp.max(lmask, axis=0, keepdims=True)
        amax = jnp.where(amax > -1e29, amax, 0.0)
        ex = jnp.where(maskT, jnp.exp(logit - amax), 0.0)
        den = jnp.sum(ex, axis=0, keepdims=True)
        alphaT = ex / (den + 1e-16)

        out_e = inv_de * jnp.dot(alphaT, xl,
                                 preferred_element_type=f32)
        g = jax.lax.dot_general(incT, alphaT * inv_dn, _DNT,
                                preferred_element_type=f32)
        hf = jnp.dot(g, out_e, preferred_element_type=f32)
        hf = hf + deg_e * bc[...]
        part = jnp.dot(hf, wo[...], preferred_element_type=f32)
        res = part if res is None else res + part

    out_ref[...] = res + bout_ref[...]


def kernel(input_features, incidence_matrix, W_enc, b_enc, W_attr, b_attr,
           W_conv, att, b_conv, W_out, b_out):
    n, _ = input_features.shape
    m = incidence_matrix.shape[1]
    emb = W_enc.shape[1]
    heads = att.shape[1]
    conv = att.shape[2] // 2

    incT = incidence_matrix.T
    args = [input_features, incT, W_enc, b_enc.reshape(1, emb),
            W_attr, b_attr.reshape(1, emb)]
    args += [W_conv[:, h * conv:(h + 1) * conv] for h in range(heads)]
    args += [att[0, h:h + 1, :conv] for h in range(heads)]
    args += [att[0, h:h + 1, conv:] for h in range(heads)]
    args += [b_conv[h * conv:(h + 1) * conv].reshape(1, conv)
             for h in range(heads)]
    args += [W_out[h * conv:(h + 1) * conv] for h in range(heads)]
    args += [b_out.reshape(1, b_out.shape[0])]

    return pl.pallas_call(
        _mushin_body,
        out_shape=jax.ShapeDtypeStruct((m, b_out.shape[0]), jnp.float32),
    )(*args)
